# baseline (speedup 1.0000x reference)
import functools

import jax
import jax.numpy as jnp
from jax import lax
from jax.experimental import pallas as pl
from jax.experimental.pallas import tpu as pltpu
from jax.experimental.pallas import tpu_sc as plsc

NUM_EXPERTS = 64
TOP_K = 8
AUX_COEF = 0.01
BT = 1024
NEG = -1.0


def _matmul_body(w_ref, h_ref, out_ref):
    out_ref[...] = jax.lax.dot_general(
        w_ref[...], h_ref[...],
        dimension_numbers=(((1,), (1,)), ((), ())),
        preferred_element_type=jnp.float32,
    )


def _aux_body(num_tokens, c_ref, p_ref, aux_ref):
    counts = jnp.sum(c_ref[...], axis=(0, 2))
    probs = jnp.sum(p_ref[...], axis=(0, 2))
    scale = AUX_COEF * NUM_EXPERTS / (num_tokens * float(num_tokens))
    aux_ref[...] = jnp.sum(counts * probs, keepdims=True).reshape(1, 1) * scale


def _sc_routing(logits, num_tokens):
    info = plsc.get_sparse_core_info()
    nw = info.num_cores * info.num_subcores
    tpw = num_tokens // nw
    ngrp = tpw // 16
    mesh = plsc.VectorSubcoreMesh(core_axis_name="c", subcore_axis_name="s")

    @functools.partial(
        pl.kernel, mesh=mesh,
        compiler_params=pltpu.CompilerParams(use_tc_tiling_on_sc=False,
                                             needs_layout_passes=False),
        out_type=[
            jax.ShapeDtypeStruct((TOP_K, num_tokens), jnp.float32),
            jax.ShapeDtypeStruct((TOP_K, num_tokens), jnp.int32),
            jax.ShapeDtypeStruct((nw, NUM_EXPERTS, 16), jnp.float32),
            jax.ShapeDtypeStruct((nw, NUM_EXPERTS, 16), jnp.float32),
        ],
        scratch_types=[
            pltpu.VMEM((NUM_EXPERTS, tpw), jnp.float32),
            pltpu.VMEM((TOP_K, tpw), jnp.float32),
            pltpu.VMEM((TOP_K, tpw), jnp.int32),
            pltpu.VMEM((NUM_EXPERTS, 16), jnp.float32),
            pltpu.VMEM((NUM_EXPERTS, 16), jnp.float32),
        ],
    )
    def k(logits_hbm, outw_hbm, outi_hbm, cnt_hbm, prob_hbm,
          buf, outw, outi, cnt, pacc):
        wid = lax.axis_index("s") * info.num_cores + lax.axis_index("c")
        base = wid * tpw
        pltpu.sync_copy(logits_hbm.at[:, pl.ds(base, tpw)], buf)
        zeros16 = jnp.zeros((16,), jnp.float32)
        for e in range(NUM_EXPERTS):
            cnt[e, :] = zeros16
            pacc[e, :] = zeros16
        iota16 = lax.iota(jnp.int32, 16)
        ones16 = jnp.ones((16,), jnp.float32)

        def group(g, carry):
            sl = pl.ds(g * 16, 16)
            lanes = iota16 + g * 16
            m0 = buf[0, sl]
            for e in range(1, NUM_EXPERTS):
                m0 = jnp.maximum(m0, buf[e, sl])
            s = jnp.zeros((16,), jnp.float32)
            for e in range(NUM_EXPERTS):
                x = jnp.exp(buf[e, sl] - m0)
                buf[e, sl] = x
                s = s + x
            rs = ones16 / s
            for e in range(NUM_EXPERTS):
                pacc[e, :] = pacc[e, :] + buf[e, sl] * rs
            vals, idxs = [], []
            for _ in range(TOP_K):
                m = buf[0, sl]
                mi = jnp.zeros((16,), jnp.int32)
                for e in range(1, NUM_EXPERTS):
                    v = buf[e, sl]
                    gt = v > m
                    m = jnp.where(gt, v, m)
                    mi = jnp.where(gt, e, mi)
                vals.append(m)
                idxs.append(mi)
                plsc.store_scatter(buf, [mi, lanes],
                                   jnp.full((16,), NEG, jnp.float32))
                plsc.addupdate_scatter(cnt, [mi, iota16], ones16)
            wsum = vals[0]
            for v in vals[1:]:
                wsum = wsum + v
            rw = ones16 / wsum
            for t in range(TOP_K):
                outw[t, sl] = vals[t] * rw
                outi[t, sl] = idxs[t]
            return carry

        lax.fori_loop(0, ngrp, group, 0)
        pltpu.sync_copy(outw, outw_hbm.at[:, pl.ds(base, tpw)])
        pltpu.sync_copy(outi, outi_hbm.at[:, pl.ds(base, tpw)])
        pltpu.sync_copy(cnt, cnt_hbm.at[wid])
        pltpu.sync_copy(pacc, prob_hbm.at[wid])

    return k(logits)


@jax.jit
def kernel(hidden_states, gate_w):
    batch, seq, hidden = hidden_states.shape
    num_tokens = batch * seq
    h_flat = hidden_states.reshape(num_tokens, hidden)
    nblk = num_tokens // BT

    logits = pl.pallas_call(
        _matmul_body,
        grid=(nblk,),
        in_specs=[
            pl.BlockSpec((NUM_EXPERTS, hidden), lambda i: (0, 0)),
            pl.BlockSpec((BT, hidden), lambda i: (i, 0)),
        ],
        out_specs=pl.BlockSpec((NUM_EXPERTS, BT), lambda i: (0, i)),
        out_shape=jax.ShapeDtypeStruct((NUM_EXPERTS, num_tokens), jnp.float32),
    )(gate_w, h_flat)

    out_w_t, out_i_t, cnt, prob = _sc_routing(logits, num_tokens)

    aux = pl.pallas_call(
        functools.partial(_aux_body, num_tokens),
        out_shape=jax.ShapeDtypeStruct((1, 1), jnp.float32),
    )(cnt, prob)

    return (out_w_t.T.reshape(batch, seq, TOP_K),
            out_i_t.T.reshape(batch, seq, TOP_K),
            aux.reshape(()))


# speedup vs baseline: 2.3265x; 2.3265x over previous
import functools

import jax
import jax.numpy as jnp
from jax.experimental import pallas as pl
from jax.experimental.pallas import tpu as pltpu

NUM_EXPERTS = 64
TOP_K = 8
AUX_COEF = 0.01
BT = 1024
NEG = -1e30


def _router_body(nblk, num_tokens, w_ref, h_ref, out_w_ref,
                 out_i_ref, aux_ref, counts_acc, probs_acc):
    i = pl.program_id(0)

    logits = jax.lax.dot_general(
        w_ref[...], h_ref[...],
        dimension_numbers=(((1,), (1,)), ((), ())),
        preferred_element_type=jnp.float32,
    )

    m = jnp.max(logits, axis=0, keepdims=True)
    ex = jnp.exp(logits - m)
    probs = ex / jnp.sum(ex, axis=0, keepdims=True)
    prob_part = jnp.sum(probs, axis=1, keepdims=True)

    iota = jax.lax.broadcasted_iota(jnp.int32, logits.shape, 0)

    work = logits
    vals, idxs = [], []
    for _ in range(TOP_K):
        vmax = jnp.max(work, axis=0, keepdims=True)
        sel = jnp.where(work == vmax, iota, NUM_EXPERTS)
        imin = jnp.min(sel, axis=0, keepdims=True)
        vals.append(vmax)
        idxs.append(imin)
        work = jnp.where(iota == imin, NEG, work)

    top_vals = jnp.concatenate(vals, axis=0)
    top_idx = jnp.concatenate(idxs, axis=0)
    exw = jnp.exp(top_vals - top_vals[0:1, :])
    out_w_ref[...] = exw / jnp.sum(exw, axis=0, keepdims=True)
    out_i_ref[...] = top_idx

    counts = jnp.sum((work == NEG).astype(jnp.float32), axis=1,
                     keepdims=True)

    @pl.when(i == 0)
    def _init():
        counts_acc[...] = counts
        probs_acc[...] = prob_part

    @pl.when(i > 0)
    def _accum():
        counts_acc[...] += counts
        probs_acc[...] += prob_part

    @pl.when(i == nblk - 1)
    def _finalize():
        scale = AUX_COEF * NUM_EXPERTS / (num_tokens * float(num_tokens))
        aux_ref[...] = jnp.sum(counts_acc[...] * probs_acc[...],
                               keepdims=True).reshape(1, 1) * scale


@jax.jit
def kernel(hidden_states, gate_w):
    batch, seq, hidden = hidden_states.shape
    num_tokens = batch * seq
    h_flat = hidden_states.reshape(num_tokens, hidden)
    nblk = num_tokens // BT

    out_w_t, out_i_t, aux = pl.pallas_call(
        functools.partial(_router_body, nblk, num_tokens),
        grid=(nblk,),
        in_specs=[
            pl.BlockSpec((NUM_EXPERTS, hidden), lambda i: (0, 0)),
            pl.BlockSpec((BT, hidden), lambda i: (i, 0)),
        ],
        out_specs=[
            pl.BlockSpec((TOP_K, BT), lambda i: (0, i)),
            pl.BlockSpec((TOP_K, BT), lambda i: (0, i)),
            pl.BlockSpec((1, 1), lambda i: (0, 0)),
        ],
        out_shape=[
            jax.ShapeDtypeStruct((TOP_K, num_tokens), jnp.float32),
            jax.ShapeDtypeStruct((TOP_K, num_tokens), jnp.int32),
            jax.ShapeDtypeStruct((1, 1), jnp.float32),
        ],
        scratch_shapes=[
            pltpu.VMEM((NUM_EXPERTS, 1), jnp.float32),
            pltpu.VMEM((NUM_EXPERTS, 1), jnp.float32),
        ],
    )(gate_w, h_flat)

    return (out_w_t.T.reshape(batch, seq, TOP_K),
            out_i_t.T.reshape(batch, seq, TOP_K),
            aux.reshape(()))
